# baseline (speedup 1.0000x reference)
import functools
import math

import jax
import jax.numpy as jnp
from jax import lax
from jax.experimental import pallas as pl
from jax.experimental.pallas import tpu as pltpu
from jax.experimental.pallas import tpu_sc as plsc

VOCAB = 1_000_000
EMB = 64
SCALE = math.sqrt(EMB)

NC = 2
NS = 16
NW = NC * NS
LANES = 16

ROWS_PER_STREAM = 128


def _make_kernel(n_tokens: int):
    assert n_tokens % (NW * ROWS_PER_STREAM) == 0
    streams_per_worker = n_tokens // (NW * ROWS_PER_STREAM)
    per_worker = streams_per_worker * ROWS_PER_STREAM

    mesh = plsc.VectorSubcoreMesh(core_axis_name="c", subcore_axis_name="s")

    @functools.partial(
        pl.kernel,
        out_type=jax.ShapeDtypeStruct((n_tokens, EMB), jnp.float32),
        mesh=mesh,
        scratch_types=[
            pltpu.VMEM((streams_per_worker, ROWS_PER_STREAM), jnp.int32),
            pltpu.VMEM((ROWS_PER_STREAM, EMB), jnp.float32),
            pltpu.SemaphoreType.DMA,
        ],
        compiler_params=pltpu.CompilerParams(use_tc_tiling_on_sc=False),
    )
    def k(tok_hbm, table_hbm, out_hbm, idx_v, rows_v, sem):
        wid = lax.axis_index("s") * NC + lax.axis_index("c")
        base = wid * per_worker
        pltpu.sync_copy(tok_hbm.at[wid], idx_v)

        @pl.loop(0, streams_per_worker)
        def _(r):
            pltpu.async_copy(table_hbm.at[idx_v.at[r]], rows_v, sem).wait()

            @pl.loop(0, ROWS_PER_STREAM)
            def _(i):
                for d in range(EMB // LANES):
                    sl = pl.ds(d * LANES, LANES)
                    rows_v[i, sl] = rows_v[i, sl] * SCALE

            pltpu.sync_copy(rows_v, out_hbm.at[pl.ds(base + r * ROWS_PER_STREAM,
                                                     ROWS_PER_STREAM)])

    return k


@jax.jit
def kernel(tokens, table):
    b, l = tokens.shape
    n = b * l
    tok = jnp.reshape(tokens, (NW, n // (NW * ROWS_PER_STREAM), ROWS_PER_STREAM))
    out = _make_kernel(n)(tok, table)
    return jnp.reshape(out, (b, l, EMB))


# speedup vs baseline: 1.2038x; 1.2038x over previous
import functools
import math

import jax
import jax.numpy as jnp
from jax import lax
from jax.experimental import pallas as pl
from jax.experimental.pallas import tpu as pltpu
from jax.experimental.pallas import tpu_sc as plsc

VOCAB = 1_000_000
EMB = 64
SCALE = math.sqrt(EMB)

NC = 2
NS = 16
NW = NC * NS
LANES = 16

ROWS_PER_STREAM = 128
GROUP = 4
GROUP_ROWS = GROUP * ROWS_PER_STREAM


def _make_kernel(n_tokens: int):
    assert n_tokens % (NW * GROUP_ROWS) == 0
    streams_per_worker = n_tokens // (NW * ROWS_PER_STREAM)
    groups_per_worker = streams_per_worker // GROUP
    per_worker = streams_per_worker * ROWS_PER_STREAM
    assert groups_per_worker % 2 == 0

    mesh = plsc.VectorSubcoreMesh(core_axis_name="c", subcore_axis_name="s")

    @functools.partial(
        pl.kernel,
        out_type=jax.ShapeDtypeStruct((n_tokens, EMB), jnp.float32),
        mesh=mesh,
        scratch_types=[
            pltpu.VMEM((streams_per_worker, ROWS_PER_STREAM), jnp.int32),
            pltpu.VMEM((GROUP_ROWS, EMB), jnp.float32),
            pltpu.VMEM((GROUP_ROWS, EMB), jnp.float32),
            pltpu.SemaphoreType.DMA,
            pltpu.SemaphoreType.DMA,
            pltpu.SemaphoreType.DMA,
            pltpu.SemaphoreType.DMA,
        ],
        compiler_params=pltpu.CompilerParams(use_tc_tiling_on_sc=False),
    )
    def k(tok_hbm, table_hbm, out_hbm, idx_v, buf0, buf1,
          sem_g0, sem_g1, sem_o0, sem_o1):
        wid = lax.axis_index("s") * NC + lax.axis_index("c")
        base = wid * per_worker

        def gather_descs(g, buf, sem):
            return [
                pltpu.make_async_copy(
                    table_hbm.at[idx_v.at[g * GROUP + j]],
                    buf.at[pl.ds(j * ROWS_PER_STREAM, ROWS_PER_STREAM)],
                    sem)
                for j in range(GROUP)
            ]

        def out_desc(g, buf, sem):
            return pltpu.make_async_copy(
                buf, out_hbm.at[pl.ds(base + g * GROUP_ROWS, GROUP_ROWS)], sem)

        pltpu.sync_copy(tok_hbm.at[wid], idx_v)
        for d in gather_descs(0, buf0, sem_g0):
            d.start()

        @pl.loop(0, groups_per_worker, step=2)
        def _(g0):
            for b in range(2):
                g = g0 + b
                buf, sem_g, sem_o = (buf0, sem_g0, sem_o0) if b == 0 else (
                    buf1, sem_g1, sem_o1)
                obuf, osem_g, osem_o = (buf1, sem_g1, sem_o1) if b == 0 else (
                    buf0, sem_g0, sem_o0)

                @pl.when(g + 1 < groups_per_worker)
                def _():
                    @pl.when(g >= 1)
                    def _():
                        out_desc(g - 1, obuf, osem_o).wait()

                    for d in gather_descs(g + 1, obuf, osem_g):
                        d.start()

                for d in gather_descs(g, buf, sem_g):
                    d.wait()

                @plsc.parallel_loop(0, GROUP_ROWS, unroll=4)
                def _(i):
                    for d in range(EMB // LANES):
                        sl = pl.ds(d * LANES, LANES)
                        buf[i, sl] = buf[i, sl] * SCALE

                out_desc(g, buf, sem_o).start()

        out_desc(groups_per_worker - 2, buf0, sem_o0).wait()
        out_desc(groups_per_worker - 1, buf1, sem_o1).wait()

    return k


@jax.jit
def kernel(tokens, table):
    b, l = tokens.shape
    n = b * l
    tok = jnp.reshape(tokens, (NW, n // (NW * ROWS_PER_STREAM), ROWS_PER_STREAM))
    out = _make_kernel(n)(tok, table)
    return jnp.reshape(out, (b, l, EMB))


# speedup vs baseline: 2.0100x; 1.6697x over previous
import functools
import math

import jax
import jax.numpy as jnp
from jax import lax
from jax.experimental import pallas as pl
from jax.experimental.pallas import tpu as pltpu
from jax.experimental.pallas import tpu_sc as plsc

VOCAB = 1_000_000
EMB = 64
B = 16384
L = 50
SCALE = math.sqrt(EMB)

NC = 2
NS = 16
LANES = 16

E_PER_CORE = EMB // NC
B_PER_SUB = B // NS
CHUNK = 128
N_CHUNK = B_PER_SUB // CHUNK


def _make_kernel():
    mesh = plsc.VectorSubcoreMesh(core_axis_name="c", subcore_axis_name="s")

    @functools.partial(
        pl.kernel,
        out_type=jax.ShapeDtypeStruct((L, EMB, B), jnp.float32),
        mesh=mesh,
        scratch_types=[
            pltpu.VMEM((L, B_PER_SUB), jnp.int32),
            pltpu.VMEM((B_PER_SUB,), jnp.float32),
            pltpu.VMEM((B_PER_SUB,), jnp.float32),
            pltpu.VMEM_SHARED((VOCAB,), jnp.float32),
            pltpu.SemaphoreType.DMA,
            pltpu.SemaphoreType.DMA,
            pltpu.SemaphoreType.DMA,
            pltpu.SemaphoreType.DMA,
            pltpu.SemaphoreType.DMA,
            pltpu.SemaphoreType.DMA,
        ],
        compiler_params=pltpu.CompilerParams(use_tc_tiling_on_sc=True),
    )
    def k(tok_hbm, tab_hbm, out_hbm, tok_v, gbuf0, gbuf1, row,
          sem_t, sem_r, sem_g0, sem_g1, sem_o0, sem_o1):
        cid = lax.axis_index("c")
        sid = lax.axis_index("s")
        e_base = cid * E_PER_CORE
        b_base = sid * B_PER_SUB

        gbufs = (gbuf0, gbuf1)
        sem_gs = (sem_g0, sem_g1)
        sem_os = (sem_o0, sem_o1)

        pltpu.make_async_copy(tok_hbm.at[sid], tok_v, sem_t).start()

        def row_desc(k_e):
            return pltpu.make_async_copy(tab_hbm.at[e_base + k_e], row, sem_r)

        def gather_descs(row_buf, l, gbuf, sem):
            return [
                pltpu.make_async_copy(
                    row_buf.at[tok_v.at[l, pl.ds(j * CHUNK, CHUNK)]],
                    gbuf.at[pl.ds(j * CHUNK, CHUNK)],
                    sem)
                for j in range(N_CHUNK)
            ]

        def out_desc(l, k_e, gbuf, sem):
            return pltpu.make_async_copy(
                gbuf, out_hbm.at[l, e_base + k_e, pl.ds(b_base, B_PER_SUB)],
                sem)

        @pl.when(sid == 0)
        def _():
            row_desc(0).start()

        pltpu.make_async_copy(tok_hbm.at[sid], tok_v, sem_t).wait()

        @pl.loop(0, E_PER_CORE)
        def _(k_e):
            @pl.when(sid == 0)
            def _():
                row_desc(k_e).wait()

            plsc.subcore_barrier()

            for d in gather_descs(row, 0, gbufs[0], sem_gs[0]):
                d.start()

            @pl.loop(0, L, step=2)
            def _(l0):
                for b2 in range(2):
                    l = l0 + b2
                    gbuf, sem_g = gbufs[b2], sem_gs[b2]
                    obuf, osem_g = gbufs[1 - b2], sem_gs[1 - b2]

                    @pl.when(l + 1 < L)
                    def _():
                        @pl.when(l >= 1)
                        def _():
                            out_desc(l - 1, k_e, obuf, sem_os[1 - b2]).wait()

                        for d in gather_descs(row, l + 1, obuf, osem_g):
                            d.start()

                    for d in gather_descs(row, l, gbuf, sem_g):
                        d.wait()

                    @plsc.parallel_loop(0, B_PER_SUB, step=LANES, unroll=4)
                    def _(i):
                        sl = pl.ds(i, LANES)
                        gbuf[sl] = gbuf[sl] * SCALE

                    out_desc(l, k_e, gbuf, sem_os[b2]).start()

            out_desc(L - 2, k_e, gbufs[0], sem_os[0]).wait()
            out_desc(L - 1, k_e, gbufs[1], sem_os[1]).wait()

            plsc.subcore_barrier()

            @pl.when((sid == 0) & (k_e + 1 < E_PER_CORE))
            def _():
                row_desc(k_e + 1).start()

    return k


_K = _make_kernel()


@jax.jit
def kernel(tokens, table):
    tok_t = jnp.transpose(tokens)
    tok_blk = jnp.transpose(jnp.reshape(tok_t, (L, NS, B_PER_SUB)), (1, 0, 2))
    tab_t = jnp.transpose(table)
    out_k = _K(tok_blk, tab_t)
    return jnp.transpose(out_k, (2, 0, 1))
